# Initial kernel scaffold; baseline (speedup 1.0000x reference)
import jax
import jax.numpy as jnp
from jax.experimental import pallas as pl


def kernel(n_input, chans_table, insts_table, rest_table):
    raise NotImplementedError("write your pallas kernel here")


# speedup vs baseline: 6.0769x; 6.0769x over previous
import functools

import jax
import jax.numpy as jnp
from jax import lax
from jax.experimental import pallas as pl
from jax.experimental.pallas import tpu as pltpu
from jax.experimental.pallas import tpu_sc as plsc

DIM = 64
N_CHANS = 16
N_INSTS = 128
SMALL = N_CHANS * N_INSTS
L = 16

NC, NS = 2, 16
NW = NC * NS

CHUNK = 640
SUB = 128
K = CHUNK // SUB


def _embed_body(idx_hbm, chans_hbm, insts_hbm, rest_hbm, out_hbm,
                chans_v, insts_v, idx_v, ridx_v, rows_v,
                gsem0, gsem1, osem0, osem1):
    wid = lax.axis_index("s") * NC + lax.axis_index("c")
    tokens = idx_hbm.shape[0]
    per_w = tokens // NW
    n_chunks = per_w // CHUNK
    base_w = wid * per_w
    gsems = (gsem0, gsem1)
    osems = (osem0, osem1)

    pltpu.sync_copy(chans_hbm, chans_v)
    pltpu.sync_copy(insts_hbm, insts_v)

    def start(ch):
        b = ch % 2
        base = base_w + ch * CHUNK
        pltpu.sync_copy(idx_hbm.at[pl.ds(base, CHUNK)], idx_v.at[b])

        def ridx_g(g, c2):
            n = idx_v[b, pl.ds(g * L, L)]
            r = jnp.maximum(n - SMALL, 0)
            j = r // TCW
            u = r - j * TCW
            h = jnp.where(u >= TCW // 2, 1, 0)
            p = TCW * j + 2 * (u - (TCW // 2) * h) + h
            row = g // (SUB // L)
            col = (g % (SUB // L)) * L
            ridx_v[b, row, pl.ds(col, L)] = p
            return c2

        lax.fori_loop(0, CHUNK // L, ridx_g, 0)
        return [
            pltpu.async_copy(rest_hbm.at[ridx_v.at[b, j]],
                             rows_v.at[b, pl.ds(j * SUB, SUB)], gsems[b])
            for j in range(K)
        ]

    def finish(ch, gcopies):
        b = ch % 2
        base = base_w + ch * CHUNK
        for cpy in gcopies:
            cpy.wait()

        def fix_g(g, c2):
            n = idx_v[b, pl.ds(g * L, L)]
            m = n < SMALL

            @pl.when(jnp.sum(m.astype(jnp.int32)) > 0)
            def _():
                ch_id = jnp.bitwise_and(n, N_CHANS - 1)
                in_id = jnp.minimum(jnp.right_shift(n, 4), N_INSTS - 1)
                toks = g * L + lax.iota(jnp.int32, L)

                def col_body(col, c3):
                    cols = jnp.full((L,), col, jnp.int32)
                    v = (plsc.load_gather(chans_v, [ch_id, cols]) +
                         plsc.load_gather(insts_v, [in_id, cols]))
                    plsc.store_scatter(rows_v.at[b], [toks, cols], v, mask=m)
                    return c3

                lax.fori_loop(0, DIM, col_body, 0)

            return c2

        lax.fori_loop(0, CHUNK // L, fix_g, 0)
        return pltpu.async_copy(rows_v.at[b], out_hbm.at[pl.ds(base, CHUNK)],
                                osems[b])

    gcopies = {}
    ocopies = {}
    for ch in range(n_chunks):
        if ch >= 2:
            ocopies[ch - 2].wait()
        gcopies[ch] = start(ch)
        if ch >= 1:
            ocopies[ch - 1] = finish(ch - 1, gcopies.pop(ch - 1))
    ocopies[n_chunks - 1] = finish(n_chunks - 1, gcopies.pop(n_chunks - 1))
    ocopies[n_chunks - 2].wait()
    ocopies[n_chunks - 1].wait()


TCW = 33280


def _transpose_body(x_ref, o_ref):
    y = x_ref[...].T
    o_ref[...] = jnp.concatenate([y[: TCW // 2], y[TCW // 2:]], axis=1)


def _to_row_major(rest_table):
    nrest = rest_table.shape[0]
    rest_t = rest_table.T
    grid = (nrest + TCW - 1) // TCW
    pairs = pl.pallas_call(
        _transpose_body,
        grid=(grid,),
        in_specs=[pl.BlockSpec((DIM, TCW), lambda j: (0, j))],
        out_specs=pl.BlockSpec((TCW // 2, 2 * DIM), lambda j: (j, 0)),
        out_shape=jax.ShapeDtypeStruct((grid * (TCW // 2), 2 * DIM),
                                       jnp.float32),
    )(rest_t)
    return pairs.reshape(grid * TCW, DIM)


_SCRATCH = [
    pltpu.VMEM((N_CHANS, DIM), jnp.float32),
    pltpu.VMEM((N_INSTS, DIM), jnp.float32),
    pltpu.VMEM((2, CHUNK), jnp.int32),
    pltpu.VMEM((2, K, SUB), jnp.int32),
    pltpu.VMEM((2, CHUNK, DIM), jnp.float32),
    pltpu.SemaphoreType.DMA,
    pltpu.SemaphoreType.DMA,
    pltpu.SemaphoreType.DMA,
    pltpu.SemaphoreType.DMA,
]


@jax.jit
def kernel(n_input, chans_table, insts_table, rest_table):
    b, s = n_input.shape
    tokens = b * s
    assert tokens % (NW * CHUNK) == 0
    idx = n_input.reshape(tokens)
    rest_rm = _to_row_major(rest_table)
    run = pl.kernel(
        _embed_body,
        out_type=jax.ShapeDtypeStruct((tokens, DIM), jnp.float32),
        compiler_params=pltpu.CompilerParams(needs_layout_passes=False,
                                             use_tc_tiling_on_sc=False),
        mesh=plsc.VectorSubcoreMesh(core_axis_name="c", subcore_axis_name="s"),
        scratch_types=_SCRATCH,
    )
    out = run(idx, chans_table, insts_table, rest_rm)
    return out.reshape(b, s, DIM)
